# Initial kernel scaffold; baseline (speedup 1.0000x reference)
import jax
import jax.numpy as jnp
from jax.experimental import pallas as pl


def kernel(c_in, n_in, tables, W, b):
    raise NotImplementedError("write your pallas kernel here")


# speedup vs baseline: 8.1883x; 8.1883x over previous
import jax
import jax.numpy as jnp
from jax import lax
from jax.experimental import pallas as pl
from jax.experimental.pallas import tpu as pltpu
from jax.experimental.pallas import tpu_sc as plsc

B = 16384
F = 26
V = 100000
D = 16
N = 13

NC = 2
NS = 16
L = 16
NW = NC * NS
BPW = B // NW
CH = 128
NCHUNK = BPW // CH


def _wide_sc_kernel(tables, c_in, n_in, wc, wnb, out,
                    cbuf, nbuf, wcv, wnbv, rows, outv, sem, gsem):
    wid = lax.axis_index("s") * NC + lax.axis_index("c")
    base = wid * BPW

    cps = []
    cps.append(pltpu.async_copy(wc, wcv, sem))
    cps.append(pltpu.async_copy(wnb, wnbv, sem))
    for f in range(F):
        cps.append(pltpu.async_copy(c_in.at[pl.ds(f * B + base, BPW)],
                                    cbuf.at[pl.ds(f * BPW, BPW)], sem))
    for n in range(N):
        cps.append(pltpu.async_copy(n_in.at[pl.ds(n * B + base, BPW)],
                                    nbuf.at[pl.ds(n * BPW, BPW)], sem))
    for cp in cps:
        cp.wait()

    wvecs = [wcv[pl.ds(D * f, L)] for f in range(F)]
    wn_b = [wnbv[n] for n in range(N)]
    bias = wnbv[N]
    lane_iota = lax.iota(jnp.int32, L)

    for c in range(NCHUNK):
        gps = []
        for f in range(F):
            gps.append(pltpu.async_copy(
                tables.at[f].at[cbuf.at[pl.ds(f * BPW + c * CH, CH)]],
                rows.at[pl.ds(f * CH, CH)], gsem))
        for gp in gps:
            gp.wait()

        def gbody(g, carry):
            vec = jnp.zeros((L,), jnp.float32)
            for j in range(L):
                i = g * L + j
                a = rows[i] * wvecs[0]
                for f in range(1, F):
                    a = a + rows[f * CH + i] * wvecs[f]
                for sh in (8, 4, 2, 1):
                    a = a + a.at[lane_iota ^ sh].get(mode="promise_in_bounds")
                vec = jnp.where(lane_iota == j, a, vec)
            outv[pl.ds(c * CH + g * L, L)] = vec
            return carry
        lax.fori_loop(0, CH // L, gbody, 0)

    for g in range(BPW // L):
        val = outv[pl.ds(g * L, L)] + bias
        for n in range(N):
            val = val + nbuf[pl.ds(n * BPW + g * L, L)] * wn_b[n]
        outv[pl.ds(g * L, L)] = val

    pltpu.sync_copy(outv, out.at[pl.ds(base, BPW)])


def kernel(c_in, n_in, tables, W, b):
    wflat = W[:, 0]
    wc = wflat[:F * D]
    wnb = jnp.broadcast_to(
        jnp.concatenate([wflat[F * D:], b])[:, None], (N + 1, L))
    c_flat = c_in.astype(jnp.int32).reshape(-1)
    n_flat = n_in.reshape(-1)

    mesh = plsc.VectorSubcoreMesh(core_axis_name="c", subcore_axis_name="s")
    f = pl.kernel(
        _wide_sc_kernel,
        mesh=mesh,
        compiler_params=pltpu.CompilerParams(use_tc_tiling_on_sc=False),
        out_type=jax.ShapeDtypeStruct((B,), jnp.float32),
        scratch_types=[
            pltpu.VMEM((F * BPW,), jnp.int32),
            pltpu.VMEM((N * BPW,), jnp.float32),
            pltpu.VMEM((F * D,), jnp.float32),
            pltpu.VMEM((N + 1, L), jnp.float32),
            pltpu.VMEM((F * CH, D), jnp.float32),
            pltpu.VMEM((BPW,), jnp.float32),
            pltpu.SemaphoreType.DMA,
            pltpu.SemaphoreType.DMA,
        ],
    )
    out = f(tables, c_flat, n_flat, wc, wnb)
    return out.reshape(B, 1)


# speedup vs baseline: 39.0894x; 4.7738x over previous
import jax
import jax.numpy as jnp
from jax import lax
from jax.experimental import pallas as pl
from jax.experimental.pallas import tpu as pltpu
from jax.experimental.pallas import tpu_sc as plsc

B = 16384
F = 26
V = 100000
D = 16
N = 13

NC = 2
NS = 16
L = 16
NW = NC * NS
BPW = B // NW
CH = 128
NST = (F * BPW) // CH

VP = 102400
VB = 10240
GJ = VP // VB


def _score_body(w_ref, t_ref, o_ref):
    fi = pl.program_id(0)
    x = t_ref[0]
    wv = w_ref[pl.ds(fi, 1)]
    o_ref[...] = jnp.dot(wv, x)[0]


def _score_tc(t2, w2d):
    return pl.pallas_call(
        _score_body,
        grid=(F, GJ),
        in_specs=[
            pl.BlockSpec((F, D), lambda f, j: (0, 0)),
            pl.BlockSpec((1, D, VB), lambda f, j: (f, 0, j)),
        ],
        out_specs=pl.BlockSpec((VB,), lambda f, j: (f * GJ + j)),
        out_shape=jax.ShapeDtypeStruct((F * VP,), jnp.float32),
    )(w2d, t2)


def _gather_body(s1d, c_in, n_in, wnb, out, cbuf, nbuf, gbuf, wnbv, outv,
                 sem, gsem):
    wid = lax.axis_index("s") * NC + lax.axis_index("c")
    base = wid * BPW

    cps = [pltpu.async_copy(wnb, wnbv, sem)]
    for f in range(F):
        cps.append(pltpu.async_copy(c_in.at[pl.ds(f * B + base, BPW)],
                                    cbuf.at[pl.ds(f * BPW, BPW)], sem))
    for n in range(N):
        cps.append(pltpu.async_copy(n_in.at[pl.ds(n * B + base, BPW)],
                                    nbuf.at[pl.ds(n * BPW, BPW)], sem))
    for cp in cps:
        cp.wait()

    for f in range(F):
        off = jnp.int32(f * VP)

        def abody(t, carry, f=f, off=off):
            sl = pl.ds(f * BPW + t * L, L)
            cbuf[sl] = cbuf[sl] + off
            return carry
        lax.fori_loop(0, BPW // L, abody, 0)

    def issue(i, carry):
        sl = pl.ds(i * CH, CH)
        pltpu.async_copy(s1d.at[cbuf.at[sl]], gbuf.at[sl], gsem)
        return carry
    lax.fori_loop(0, NST, issue, 0)

    def drain(i, carry):
        sl = pl.ds(i * CH, CH)
        pltpu.make_async_copy(s1d.at[cbuf.at[sl]], gbuf.at[sl], gsem).wait()
        return carry
    lax.fori_loop(0, NST, drain, 0)

    wn_b = [wnbv[n] for n in range(N)]
    bias = wnbv[N]

    def gbody(g, carry):
        val = gbuf[pl.ds(g * L, L)] + bias
        for f in range(1, F):
            val = val + gbuf[pl.ds(f * BPW + g * L, L)]
        for n in range(N):
            val = val + nbuf[pl.ds(n * BPW + g * L, L)] * wn_b[n]
        outv[pl.ds(g * L, L)] = val
        return carry
    lax.fori_loop(0, BPW // L, gbody, 0)

    pltpu.sync_copy(outv, out.at[pl.ds(base, BPW)])


def kernel(c_in, n_in, tables, W, b):
    wflat = W[:, 0]
    w2d = wflat[:F * D].reshape(F, D)
    wnb = jnp.broadcast_to(
        jnp.concatenate([wflat[F * D:], b])[:, None], (N + 1, L))
    c_flat = c_in.astype(jnp.int32).reshape(-1)
    n_flat = n_in.reshape(-1)

    t2 = jnp.transpose(tables, (0, 2, 1))
    s1d = _score_tc(t2, w2d)

    mesh = plsc.VectorSubcoreMesh(core_axis_name="c", subcore_axis_name="s")
    f = pl.kernel(
        _gather_body,
        mesh=mesh,
        compiler_params=pltpu.CompilerParams(use_tc_tiling_on_sc=False),
        out_type=jax.ShapeDtypeStruct((B,), jnp.float32),
        scratch_types=[
            pltpu.VMEM((F * BPW,), jnp.int32),
            pltpu.VMEM((N * BPW,), jnp.float32),
            pltpu.VMEM((F * BPW,), jnp.float32),
            pltpu.VMEM((N + 1, L), jnp.float32),
            pltpu.VMEM((BPW,), jnp.float32),
            pltpu.SemaphoreType.DMA,
            pltpu.SemaphoreType.DMA,
        ],
    )
    out = f(s1d, c_flat, n_flat, wnb)
    return out.reshape(B, 1)


# speedup vs baseline: 55.6146x; 1.4228x over previous
import jax
import jax.numpy as jnp
from jax import lax
from jax.experimental import pallas as pl
from jax.experimental.pallas import tpu as pltpu
from jax.experimental.pallas import tpu_sc as plsc

B = 16384
F = 26
V = 100000
D = 16
N = 13

NC = 2
NS = 16
L = 16
NW = NC * NS
BPW = B // NW
CH = 128
NST = (F * BPW) // CH

VP = 102400
VB = 20480
GJ = VP // VB


def _score_body(w_ref, t_ref, o_ref):
    fi = pl.program_id(0)
    x = t_ref[0]
    wv = w_ref[pl.ds(fi, 1)]
    o_ref[...] = jnp.dot(wv, x)[0]


def _score_tc(t2, w2d):
    return pl.pallas_call(
        _score_body,
        grid=(F, GJ),
        compiler_params=pltpu.CompilerParams(
            dimension_semantics=("parallel", "parallel")),
        in_specs=[
            pl.BlockSpec((F, D), lambda f, j: (0, 0)),
            pl.BlockSpec((1, D, VB), lambda f, j: (f, 0, j)),
        ],
        out_specs=pl.BlockSpec((VB,), lambda f, j: (f * GJ + j)),
        out_shape=jax.ShapeDtypeStruct((F * VP,), jnp.float32),
    )(w2d, t2)


def _gather_body(s1d, c_in, n_in, wnb, out, cbuf, nbuf, gbuf, wnbv, outv,
                 sem, gsem):
    wid = lax.axis_index("s") * NC + lax.axis_index("c")
    base = wid * BPW

    cps = [pltpu.async_copy(wnb, wnbv, sem)]
    for f in range(F):
        cps.append(pltpu.async_copy(c_in.at[pl.ds(f * B + base, BPW)],
                                    cbuf.at[pl.ds(f * BPW, BPW)], sem))
    for n in range(N):
        cps.append(pltpu.async_copy(n_in.at[pl.ds(n * B + base, BPW)],
                                    nbuf.at[pl.ds(n * BPW, BPW)], sem))
    for cp in cps:
        cp.wait()

    for f in range(F):
        off = jnp.int32(f * VP)

        def abody(t, carry, f=f, off=off):
            sl = pl.ds(f * BPW + t * L, L)
            cbuf[sl] = cbuf[sl] + off
            return carry
        lax.fori_loop(0, BPW // L, abody, 0)

    def issue(i, carry):
        sl = pl.ds(i * CH, CH)
        pltpu.async_copy(s1d.at[cbuf.at[sl]], gbuf.at[sl], gsem)
        return carry
    lax.fori_loop(0, NST, issue, 0)

    def drain(i, carry):
        sl = pl.ds(i * CH, CH)
        pltpu.make_async_copy(s1d.at[cbuf.at[sl]], gbuf.at[sl], gsem).wait()
        return carry
    lax.fori_loop(0, NST, drain, 0)

    wn_b = [wnbv[n] for n in range(N)]
    bias = wnbv[N]

    def gbody(g, carry):
        val = gbuf[pl.ds(g * L, L)] + bias
        for f in range(1, F):
            val = val + gbuf[pl.ds(f * BPW + g * L, L)]
        for n in range(N):
            val = val + nbuf[pl.ds(n * BPW + g * L, L)] * wn_b[n]
        outv[pl.ds(g * L, L)] = val
        return carry
    lax.fori_loop(0, BPW // L, gbody, 0)

    pltpu.sync_copy(outv, out.at[pl.ds(base, BPW)])


def kernel(c_in, n_in, tables, W, b):
    wflat = W[:, 0]
    w2d = wflat[:F * D].reshape(F, D)
    wnb = jnp.broadcast_to(
        jnp.concatenate([wflat[F * D:], b])[:, None], (N + 1, L))
    c_flat = c_in.astype(jnp.int32).reshape(-1)
    n_flat = n_in.reshape(-1)

    t2 = jnp.transpose(tables, (0, 2, 1))
    s1d = _score_tc(t2, w2d)

    mesh = plsc.VectorSubcoreMesh(core_axis_name="c", subcore_axis_name="s")
    f = pl.kernel(
        _gather_body,
        mesh=mesh,
        compiler_params=pltpu.CompilerParams(use_tc_tiling_on_sc=False),
        out_type=jax.ShapeDtypeStruct((B,), jnp.float32),
        scratch_types=[
            pltpu.VMEM((F * BPW,), jnp.int32),
            pltpu.VMEM((N * BPW,), jnp.float32),
            pltpu.VMEM((F * BPW,), jnp.float32),
            pltpu.VMEM((N + 1, L), jnp.float32),
            pltpu.VMEM((BPW,), jnp.float32),
            pltpu.SemaphoreType.DMA,
            pltpu.SemaphoreType.DMA,
        ],
    )
    out = f(s1d, c_flat, n_flat, wnb)
    return out.reshape(B, 1)


# speedup vs baseline: 86.8146x; 1.5610x over previous
import jax
import jax.numpy as jnp
from jax import lax
from jax.experimental import pallas as pl
from jax.experimental.pallas import tpu as pltpu
from jax.experimental.pallas import tpu_sc as plsc

B = 16384
F = 26
V = 100000
D = 16
N = 13

NC = 2
NS = 16
L = 16
NW = NC * NS
BPW = B // NW
CH = 128
NST = (F * BPW) // CH

VP = 102400
VB = 102400
GJ = VP // VB


def _score_body(w_ref, t_ref, o_ref):
    fi = pl.program_id(0)
    x = t_ref[0]
    wv = w_ref[pl.ds(fi, 1)]
    o_ref[...] = jnp.dot(wv, x)[0]


def _score_tc(t2, w2d):
    return pl.pallas_call(
        _score_body,
        grid=(F, GJ),
        compiler_params=pltpu.CompilerParams(
            dimension_semantics=("parallel", "parallel")),
        in_specs=[
            pl.BlockSpec((F, D), lambda f, j: (0, 0)),
            pl.BlockSpec((1, D, VB), lambda f, j: (f, 0, j)),
        ],
        out_specs=pl.BlockSpec((VB,), lambda f, j: (f * GJ + j)),
        out_shape=jax.ShapeDtypeStruct((F * VP,), jnp.float32),
    )(w2d, t2)


def _gather_body(s1d, c_in, n_in, wnb, out, cbuf, nbuf, gbuf, wnbv, outv,
                 sem, gsem):
    wid = lax.axis_index("s") * NC + lax.axis_index("c")
    base = wid * BPW

    cps = [pltpu.async_copy(wnb, wnbv, sem)]
    for f in range(F):
        cps.append(pltpu.async_copy(c_in.at[pl.ds(f * B + base, BPW)],
                                    cbuf.at[pl.ds(f * BPW, BPW)], sem))
    for n in range(N):
        cps.append(pltpu.async_copy(n_in.at[pl.ds(n * B + base, BPW)],
                                    nbuf.at[pl.ds(n * BPW, BPW)], sem))
    for cp in cps:
        cp.wait()

    for f in range(F):
        off = jnp.int32(f * VP)

        def abody(t, carry, f=f, off=off):
            sl = pl.ds(f * BPW + t * L, L)
            cbuf[sl] = cbuf[sl] + off
            return carry
        lax.fori_loop(0, BPW // L, abody, 0)

    def issue(i, carry):
        sl = pl.ds(i * CH, CH)
        pltpu.async_copy(s1d.at[cbuf.at[sl]], gbuf.at[sl], gsem)
        return carry
    lax.fori_loop(0, NST, issue, 0)

    def drain(i, carry):
        sl = pl.ds(i * CH, CH)
        pltpu.make_async_copy(s1d.at[cbuf.at[sl]], gbuf.at[sl], gsem).wait()
        return carry
    lax.fori_loop(0, NST, drain, 0)

    wn_b = [wnbv[n] for n in range(N)]
    bias = wnbv[N]

    def gbody(g, carry):
        val = gbuf[pl.ds(g * L, L)] + bias
        for f in range(1, F):
            val = val + gbuf[pl.ds(f * BPW + g * L, L)]
        for n in range(N):
            val = val + nbuf[pl.ds(n * BPW + g * L, L)] * wn_b[n]
        outv[pl.ds(g * L, L)] = val
        return carry
    lax.fori_loop(0, BPW // L, gbody, 0)

    pltpu.sync_copy(outv, out.at[pl.ds(base, BPW)])


def kernel(c_in, n_in, tables, W, b):
    wflat = W[:, 0]
    w2d = wflat[:F * D].reshape(F, D)
    wnb = jnp.broadcast_to(
        jnp.concatenate([wflat[F * D:], b])[:, None], (N + 1, L))
    c_flat = c_in.astype(jnp.int32).reshape(-1)
    n_flat = n_in.reshape(-1)

    t2 = jnp.transpose(tables, (0, 2, 1))
    s1d = _score_tc(t2, w2d)

    mesh = plsc.VectorSubcoreMesh(core_axis_name="c", subcore_axis_name="s")
    f = pl.kernel(
        _gather_body,
        mesh=mesh,
        compiler_params=pltpu.CompilerParams(use_tc_tiling_on_sc=False),
        out_type=jax.ShapeDtypeStruct((B,), jnp.float32),
        scratch_types=[
            pltpu.VMEM((F * BPW,), jnp.int32),
            pltpu.VMEM((N * BPW,), jnp.float32),
            pltpu.VMEM((F * BPW,), jnp.float32),
            pltpu.VMEM((N + 1, L), jnp.float32),
            pltpu.VMEM((BPW,), jnp.float32),
            pltpu.SemaphoreType.DMA,
            pltpu.SemaphoreType.DMA,
        ],
    )
    out = f(s1d, c_flat, n_flat, wnb)
    return out.reshape(B, 1)
